# baseline (speedup 1.0000x reference)
import functools

import jax
import jax.numpy as jnp
from jax import lax
from jax.experimental import pallas as pl
from jax.experimental.pallas import tpu as pltpu
from jax.experimental.pallas import tpu_sc as plsc

D_MODEL = 1024
SEQ_LEN = 4096

_NC = 2
_NS = 16
_NW = _NC * _NS
_B_PER_W = SEQ_LEN // _NW
_CHUNK = 16
_NCHUNK = _B_PER_W // _CHUNK
_NSPR = 7
_NB = 3
_RING = 4


def _embed_body(table_hbm, idx_hbm, out_hbm, idx_v,
                b0, b1, b2, sh,
                sg0, sg1, sg2, sx0, sx1, sx2, sx3,
                sd0, sd1, sd2, sd3, sdir):
    bufs = (b0, b1, b2)
    sgs = (sg0, sg1, sg2)
    sxs = (sx0, sx1, sx2, sx3)
    sds = (sd0, sd1, sd2, sd3)
    wid = lax.axis_index("s") * _NC + lax.axis_index("c")
    sid = lax.axis_index("s")
    base = wid * _B_PER_W
    pltpu.sync_copy(idx_hbm.at[pl.ds(base, _B_PER_W)], idx_v)

    def start_g(c):
        return pltpu.async_copy(
            table_hbm.at[idx_v.at[pl.ds(c * _CHUNK, _CHUNK)]],
            bufs[c % _NB], sgs[c % _NB])

    def start_x(c):
        return pltpu.async_copy(
            bufs[c % _NB], sh.at[sid].at[c % _RING], sxs[c % _RING])

    def start_d(c):
        return pltpu.async_copy(
            sh.at[sid].at[c % _RING],
            out_hbm.at[pl.ds(base + c * _CHUNK, _CHUNK)], sds[c % _RING])

    gathers = [start_g(0), start_g(1), start_g(2)]
    drains = {}
    for c in range(_NSPR):
        if c >= _RING:
            drains[c - _RING].wait()
        gathers[c].wait()
        start_x(c).wait()
        drains[c] = start_d(c)
        if c + _NB < _NCHUNK:
            gathers.append(start_g(c + _NB))
    gathers[_NCHUNK - 1].wait()
    tail = pltpu.async_copy(
        bufs[(_NCHUNK - 1) % _NB],
        out_hbm.at[pl.ds(base + (_NCHUNK - 1) * _CHUNK, _CHUNK)], sdir)
    for c in range(max(0, _NSPR - _RING), _NSPR):
        drains[c].wait()
    tail.wait()


_embed = functools.partial(
    pl.kernel,
    mesh=plsc.VectorSubcoreMesh(core_axis_name="c", subcore_axis_name="s"),
    out_type=jax.ShapeDtypeStruct((SEQ_LEN, D_MODEL), jnp.float32),
    scratch_types=(
        [pltpu.VMEM((_B_PER_W,), jnp.int32)]
        + [pltpu.VMEM((_CHUNK, D_MODEL), jnp.float32) for _ in range(_NB)]
        + [pltpu.VMEM_SHARED((_NS, _RING, _CHUNK, D_MODEL), jnp.float32)]
        + [pltpu.SemaphoreType.DMA for _ in range(_NB + 2 * _RING + 1)]
    ),
)(_embed_body)


@jax.jit
def kernel(tokens, W_E):
    return _embed(W_E, tokens.astype(jnp.int32))
